# baseline (speedup 1.0000x reference)
import functools

import jax
import jax.numpy as jnp
from jax import lax
from jax.experimental import pallas as pl
from jax.experimental.pallas import tpu as pltpu
from jax.experimental.pallas import tpu_sc as plsc

N = 10000
E = 320000
D = 128
DE = 16
H = 40
HP = 48
L = 3
ALPHA = 0.5

NC = 2
NS = 16
NW = NC * NS
ET = E // NW
CK = 80
C = ET // CK
NT = N // NS

_f32 = jnp.float32


def _dot(a, b):
  return jnp.dot(a, b, preferred_element_type=_f32)


K_PIPE = 5


def _gather_body(pd_hbm, ps_hbm, dsti_hbm, srci_hbm, gt_hbm,
                 idxd_v, idxs_v, bufd, bufs, outt, semd, sems, semo):
  c = lax.axis_index("c")
  s = lax.axis_index("s")
  w = s * NC + c
  pltpu.sync_copy(dsti_hbm.at[w], idxd_v)
  pltpu.sync_copy(srci_hbm.at[w], idxs_v)
  base = w * ET
  lanes = lax.iota(jnp.int32, 16)

  NG = C // K_PIPE

  def fire(jj):
    j0 = jj * K_PIPE
    p = (jj % 2) * K_PIPE
    for b in range(K_PIPE):
      pltpu.async_copy(pd_hbm.at[idxd_v.at[j0 + b]], bufd.at[p + b], semd)
      pltpu.async_copy(ps_hbm.at[idxs_v.at[j0 + b]], bufs.at[p + b], sems)

  fire(0)

  def group(jj, carry):
    @pl.when(jj + 1 < NG)
    def _():
      fire(jj + 1)

    j0 = jj * K_PIPE
    p = (jj % 2) * K_PIPE
    for b in range(K_PIPE):
      pltpu.make_async_copy(pd_hbm.at[idxd_v.at[j0 + b]], bufd.at[p + b],
                            semd).wait()
      pltpu.make_async_copy(ps_hbm.at[idxs_v.at[j0 + b]], bufs.at[p + b],
                            sems).wait()
    for b in range(K_PIPE):
      @pl.when(jj > 0)
      def _(b=b):
        jprev = (jj - 1) * K_PIPE + b
        pltpu.make_async_copy(
            outt.at[b], gt_hbm.at[:, pl.ds(base + jprev * CK, CK)],
            semo).wait()

      def addrow(r, carry2, b=b):
        col = jnp.full((16,), r, jnp.int32)
        for t in range(HP // 16):
          sl = pl.ds(t * 16, 16)
          v = bufd[p + b, r, sl] + bufs[p + b, r, sl]
          plsc.store_scatter(outt.at[b], [lanes + (16 * t), col], v)
        return carry2

      lax.fori_loop(0, CK, addrow, 0)
      pltpu.async_copy(
          outt.at[b], gt_hbm.at[:, pl.ds(base + (j0 + b) * CK, CK)], semo)
    return carry

  lax.fori_loop(0, NG, group, 0)
  for b in range(K_PIPE):
    jprev = (NG - 1) * K_PIPE + b
    pltpu.make_async_copy(
        outt.at[b], gt_hbm.at[:, pl.ds(base + jprev * CK, CK)], semo).wait()


_gather_call = pl.kernel(
    _gather_body,
    out_type=jax.ShapeDtypeStruct((HP, E), _f32),
    mesh=plsc.VectorSubcoreMesh(core_axis_name="c", subcore_axis_name="s"),
    compiler_params=pltpu.CompilerParams(use_tc_tiling_on_sc=False,
                                         needs_layout_passes=False),
    scratch_types=[
        pltpu.VMEM((C, CK), jnp.int32),
        pltpu.VMEM((C, CK), jnp.int32),
        pltpu.VMEM((2 * K_PIPE, CK, HP), _f32),
        pltpu.VMEM((2 * K_PIPE, CK, HP), _f32),
        pltpu.VMEM((K_PIPE, HP, CK), _f32),
        pltpu.SemaphoreType.DMA,
        pltpu.SemaphoreType.DMA,
        pltpu.SemaphoreType.DMA,
    ],
)


def _scatter_body(et_hbm, dsti_hbm, part_hbm, erow_hbm, idx_v, buft, buf,
                  zbuf, seml, semsc, semo, aggr_sp):
  c = lax.axis_index("c")
  s = lax.axis_index("s")
  w = s * NC + c

  def zrow(i, carry):
    zbuf[i] = jnp.zeros((16,), _f32)
    return carry

  lax.fori_loop(0, NT, zrow, 0)
  pltpu.sync_copy(zbuf, aggr_sp.at[pl.ds(s * NT, NT)])
  pltpu.sync_copy(dsti_hbm.at[w], idx_v)
  plsc.subcore_barrier()

  base = w * ET
  lanes = lax.iota(jnp.int32, 16)
  NG = C // K_PIPE

  def fire(jj):
    j0 = jj * K_PIPE
    p = (jj % 2) * K_PIPE
    for b in range(K_PIPE):
      pltpu.async_copy(et_hbm.at[:, pl.ds(base + (j0 + b) * CK, CK)],
                       buft.at[p + b], seml)

  fire(0)

  def group(jj, carry):
    @pl.when(jj + 1 < NG)
    def _():
      fire(jj + 1)

    j0 = jj * K_PIPE
    p = (jj % 2) * K_PIPE
    for b in range(K_PIPE):
      pltpu.make_async_copy(et_hbm.at[:, pl.ds(base + (j0 + b) * CK, CK)],
                            buft.at[p + b], seml).wait()
    for b in range(K_PIPE):
      @pl.when(jj > 0)
      def _(b=b):
        jprev = (jj - 1) * K_PIPE + b
        pltpu.make_async_copy(buf.at[b], aggr_sp.at[idx_v.at[jprev]],
                              semsc).wait()
        pltpu.make_async_copy(
            buf.at[b], erow_hbm.at[pl.ds(base + jprev * CK, CK)],
            semo).wait()

      def trow(ch, carry2, b=b):
        col = jnp.full((16,), ch, jnp.int32)
        for gseg in range(CK // 16):
          v = buft[p + b, ch, pl.ds(16 * gseg, 16)]
          plsc.store_scatter(buf.at[b], [lanes + (16 * gseg), col], v)
        return carry2

      lax.fori_loop(0, DE, trow, 0)
      pltpu.async_copy(buf.at[b], aggr_sp.at[idx_v.at[j0 + b]], semsc,
                       add=True)
      pltpu.async_copy(
          buf.at[b], erow_hbm.at[pl.ds(base + (j0 + b) * CK, CK)], semo)
    return carry

  lax.fori_loop(0, NG, group, 0)
  for b in range(K_PIPE):
    jprev = (NG - 1) * K_PIPE + b
    pltpu.make_async_copy(buf.at[b], aggr_sp.at[idx_v.at[jprev]],
                          semsc).wait()
    pltpu.make_async_copy(
        buf.at[b], erow_hbm.at[pl.ds(base + jprev * CK, CK)], semo).wait()
  plsc.subcore_barrier()
  pltpu.sync_copy(aggr_sp.at[pl.ds(s * NT, NT)],
                  part_hbm.at[c, pl.ds(s * NT, NT)])


_scatter_call = pl.kernel(
    _scatter_body,
    out_type=(jax.ShapeDtypeStruct((NC, N, DE), _f32),
              jax.ShapeDtypeStruct((E, DE), _f32)),
    mesh=plsc.VectorSubcoreMesh(core_axis_name="c", subcore_axis_name="s"),
    compiler_params=pltpu.CompilerParams(use_tc_tiling_on_sc=False,
                                         needs_layout_passes=False),
    scratch_types=[
        pltpu.VMEM((C, CK), jnp.int32),
        pltpu.VMEM((2 * K_PIPE, DE, CK), _f32),
        pltpu.VMEM((K_PIPE, CK, DE), _f32),
        pltpu.VMEM((NT, DE), _f32),
        pltpu.SemaphoreType.DMA,
        pltpu.SemaphoreType.DMA,
        pltpu.SemaphoreType.DMA,
        pltpu.VMEM_SHARED((N, DE), _f32),
    ],
)


BN = 5000
BE = 64000


def _proj_body(h_ref, wd_ref, ws_ref, pd_ref, ps_ref):
  h = h_ref[...]
  pd_ref[...] = _dot(h, wd_ref[...])
  ps_ref[...] = _dot(h, ws_ref[...])


def _proj(h, wd, ws):
  return pl.pallas_call(
      _proj_body,
      grid=(N // BN,),
      in_specs=[
          pl.BlockSpec((BN, D), lambda i: (i, 0)),
          pl.BlockSpec((D, HP), lambda i: (0, 0)),
          pl.BlockSpec((D, HP), lambda i: (0, 0)),
      ],
      out_specs=[
          pl.BlockSpec((BN, HP), lambda i: (i, 0)),
          pl.BlockSpec((BN, HP), lambda i: (i, 0)),
      ],
      out_shape=[
          jax.ShapeDtypeStruct((N, HP), _f32),
          jax.ShapeDtypeStruct((N, HP), _f32),
      ],
  )(h, wd, ws)


def _edge_body(gt_ref, eat_ref, wet_ref, b1_ref, w2t_ref, b2_ref, out_ref):
  z = gt_ref[...] + _dot(wet_ref[...], eat_ref[...]) + b1_ref[...]
  z = jnp.maximum(z, 0.0)
  out_ref[...] = _dot(w2t_ref[...], z) + b2_ref[...]


def _edge_mlp(gt, eat, wet, b1, w2t, b2):
  return pl.pallas_call(
      _edge_body,
      grid=(E // BE,),
      in_specs=[
          pl.BlockSpec((HP, BE), lambda i: (0, i)),
          pl.BlockSpec((DE, BE), lambda i: (0, i)),
          pl.BlockSpec((HP, DE), lambda i: (0, 0)),
          pl.BlockSpec((HP, 1), lambda i: (0, 0)),
          pl.BlockSpec((DE, HP), lambda i: (0, 0)),
          pl.BlockSpec((DE, 1), lambda i: (0, 0)),
      ],
      out_specs=pl.BlockSpec((DE, BE), lambda i: (0, i)),
      out_shape=jax.ShapeDtypeStruct((DE, E), _f32),
  )(gt, eat, wet, b1, w2t, b2)


def _node_body(h_ref, part_ref, w1h_ref, w1a_ref, b1_ref, w2_ref, b2_ref,
               wd_ref, ws_ref, hn_ref, pd_ref, ps_ref):
  h = h_ref[...]
  aggr = part_ref[0] + part_ref[1]
  u = jnp.maximum(_dot(h, w1h_ref[...]) + _dot(aggr, w1a_ref[...])
                  + b1_ref[...], 0.0)
  delta = _dot(u, w2_ref[...]) + b2_ref[...]
  hn = ALPHA * h + (1.0 - ALPHA) * jnp.maximum(delta, 0.0)
  hn_ref[...] = hn
  pd_ref[...] = _dot(hn, wd_ref[...])
  ps_ref[...] = _dot(hn, ws_ref[...])


def _node_mlp(h, part, w1h, w1a, b1, w2, b2, wd, ws):
  return pl.pallas_call(
      _node_body,
      grid=(N // BN,),
      in_specs=[
          pl.BlockSpec((BN, D), lambda i: (i, 0)),
          pl.BlockSpec((NC, BN, DE), lambda i: (0, i, 0)),
          pl.BlockSpec((D, H), lambda i: (0, 0)),
          pl.BlockSpec((DE, H), lambda i: (0, 0)),
          pl.BlockSpec((1, H), lambda i: (0, 0)),
          pl.BlockSpec((H, D), lambda i: (0, 0)),
          pl.BlockSpec((1, D), lambda i: (0, 0)),
          pl.BlockSpec((D, HP), lambda i: (0, 0)),
          pl.BlockSpec((D, HP), lambda i: (0, 0)),
      ],
      out_specs=[
          pl.BlockSpec((BN, D), lambda i: (i, 0)),
          pl.BlockSpec((BN, HP), lambda i: (i, 0)),
          pl.BlockSpec((BN, HP), lambda i: (i, 0)),
      ],
      out_shape=[
          jax.ShapeDtypeStruct((N, D), _f32),
          jax.ShapeDtypeStruct((N, HP), _f32),
          jax.ShapeDtypeStruct((N, HP), _f32),
      ],
  )(h, part, w1h, w1a, b1, w2, b2, wd, ws)


def kernel(h, edge_index, edge_attr, Wr1, br1, Wr2, br2, Wo1, bo1, Wo2, bo2):
  src = edge_index[0]
  dst = edge_index[1]
  dst3 = dst.reshape(NW, C, CK)
  src3 = src.reshape(NW, C, CK)

  pad_c = lambda m: jnp.pad(m, ((0, 0), (0, HP - H)))
  Wd = [pad_c(Wr1[l, :D]) for l in range(L)]
  Ws = [pad_c(Wr1[l, D:2 * D]) for l in range(L)]
  WeT = [pad_c(Wr1[l, 2 * D:]).T for l in range(L)]
  B1 = [jnp.pad(br1[l].reshape(H, 1), ((0, HP - H), (0, 0))) for l in range(L)]
  W2T = [jnp.pad(Wr2[l], ((0, HP - H), (0, 0))).T for l in range(L)]
  B2 = [br2[l].reshape(DE, 1) for l in range(L)]
  W1h = [Wo1[l, :D] for l in range(L)]
  W1a = [Wo1[l, D:] for l in range(L)]
  Bo1 = [bo1[l].reshape(1, H) for l in range(L)]
  Wo2l = [Wo2[l] for l in range(L)]
  Bo2 = [bo2[l].reshape(1, D) for l in range(L)]
  zeroW = jnp.zeros((D, HP), _f32)

  pd, ps = _proj(h, Wd[0], Ws[0])
  hs = [h]
  eas = [edge_attr]
  eat = edge_attr.T
  for l in range(L):
    gt = _gather_call(pd, ps, dst3, src3)
    et = _edge_mlp(gt, eat, WeT[l], B1[l], W2T[l], B2[l])
    part, e_row = _scatter_call(et, dst3)
    wd_next = Wd[l + 1] if l + 1 < L else zeroW
    ws_next = Ws[l + 1] if l + 1 < L else zeroW
    h, pd, ps = _node_mlp(h, part, W1h[l], W1a[l], Bo1[l], Wo2l[l], Bo2[l],
                          wd_next, ws_next)
    eat = et
    hs.append(h)
    eas.append(e_row)
  return (h, jnp.stack(hs), jnp.stack(eas))


# speedup vs baseline: 1.0027x; 1.0027x over previous
import functools

import jax
import jax.numpy as jnp
from jax import lax
from jax.experimental import pallas as pl
from jax.experimental.pallas import tpu as pltpu
from jax.experimental.pallas import tpu_sc as plsc

N = 10000
E = 320000
D = 128
DE = 16
H = 40
HP = 48
L = 3
ALPHA = 0.5

NC = 2
NS = 16
NW = NC * NS
ET = E // NW
CK = 80
C = ET // CK
NT = N // NS

_f32 = jnp.float32


def _dot(a, b):
  return jnp.dot(a, b, preferred_element_type=_f32)


K_PIPE = 5


def _gather_body(pd_hbm, ps_hbm, dsti_hbm, srci_hbm, gt_hbm,
                 idxd_v, idxs_v, bufd, bufs, outt, semd, sems, semo):
  c = lax.axis_index("c")
  s = lax.axis_index("s")
  w = s * NC + c
  pltpu.sync_copy(dsti_hbm.at[w], idxd_v)
  pltpu.sync_copy(srci_hbm.at[w], idxs_v)
  base = w * ET
  lanes = lax.iota(jnp.int32, 16)

  NG = C // K_PIPE

  def fire(jj):
    j0 = jj * K_PIPE
    p = (jj % 2) * K_PIPE
    for b in range(K_PIPE):
      pltpu.async_copy(pd_hbm.at[idxd_v.at[j0 + b]], bufd.at[p + b], semd)
      pltpu.async_copy(ps_hbm.at[idxs_v.at[j0 + b]], bufs.at[p + b], sems)

  fire(0)

  def group(jj, carry):
    @pl.when(jj + 1 < NG)
    def _():
      fire(jj + 1)

    j0 = jj * K_PIPE
    p = (jj % 2) * K_PIPE
    for b in range(K_PIPE):
      pltpu.make_async_copy(pd_hbm.at[idxd_v.at[j0 + b]], bufd.at[p + b],
                            semd).wait()
      pltpu.make_async_copy(ps_hbm.at[idxs_v.at[j0 + b]], bufs.at[p + b],
                            sems).wait()
    for b in range(K_PIPE):
      @pl.when(jj > 0)
      def _(b=b):
        jprev = (jj - 1) * K_PIPE + b
        pltpu.make_async_copy(
            outt.at[b], gt_hbm.at[:, pl.ds(base + jprev * CK, CK)],
            semo).wait()

      def addrow(r, carry2, b=b):
        col = jnp.full((16,), r, jnp.int32)
        for t in range(HP // 16):
          sl = pl.ds(t * 16, 16)
          v = bufd[p + b, r, sl] + bufs[p + b, r, sl]
          plsc.store_scatter(outt.at[b], [lanes + (16 * t), col], v)
        return carry2

      lax.fori_loop(0, CK, addrow, 0)
      pltpu.async_copy(
          outt.at[b], gt_hbm.at[:, pl.ds(base + (j0 + b) * CK, CK)], semo)
    return carry

  lax.fori_loop(0, NG, group, 0)
  for b in range(K_PIPE):
    jprev = (NG - 1) * K_PIPE + b
    pltpu.make_async_copy(
        outt.at[b], gt_hbm.at[:, pl.ds(base + jprev * CK, CK)], semo).wait()


_gather_call = pl.kernel(
    _gather_body,
    out_type=jax.ShapeDtypeStruct((HP, E), _f32),
    mesh=plsc.VectorSubcoreMesh(core_axis_name="c", subcore_axis_name="s"),
    compiler_params=pltpu.CompilerParams(use_tc_tiling_on_sc=False,
                                         needs_layout_passes=False),
    scratch_types=[
        pltpu.VMEM((C, CK), jnp.int32),
        pltpu.VMEM((C, CK), jnp.int32),
        pltpu.VMEM((2 * K_PIPE, CK, HP), _f32),
        pltpu.VMEM((2 * K_PIPE, CK, HP), _f32),
        pltpu.VMEM((K_PIPE, HP, CK), _f32),
        pltpu.SemaphoreType.DMA,
        pltpu.SemaphoreType.DMA,
        pltpu.SemaphoreType.DMA,
    ],
)


def _scatter_body(et_hbm, dsti_hbm, part_hbm, erow_hbm, idx_v, buft, buf,
                  zbuf, seml, semsc, semo, aggr_sp):
  c = lax.axis_index("c")
  s = lax.axis_index("s")
  w = s * NC + c

  def zrow(i, carry):
    zbuf[i] = jnp.zeros((16,), _f32)
    return carry

  lax.fori_loop(0, NT, zrow, 0)
  pltpu.sync_copy(zbuf, aggr_sp.at[pl.ds(s * NT, NT)])
  pltpu.sync_copy(dsti_hbm.at[w], idx_v)
  plsc.subcore_barrier()

  base = w * ET
  lanes = lax.iota(jnp.int32, 16)
  NG = C // K_PIPE

  def fire(jj):
    j0 = jj * K_PIPE
    p = (jj % 2) * K_PIPE
    for b in range(K_PIPE):
      pltpu.async_copy(et_hbm.at[:, pl.ds(base + (j0 + b) * CK, CK)],
                       buft.at[p + b], seml)

  fire(0)

  def group(jj, carry):
    @pl.when(jj + 1 < NG)
    def _():
      fire(jj + 1)

    j0 = jj * K_PIPE
    p = (jj % 2) * K_PIPE
    for b in range(K_PIPE):
      pltpu.make_async_copy(et_hbm.at[:, pl.ds(base + (j0 + b) * CK, CK)],
                            buft.at[p + b], seml).wait()
    for b in range(K_PIPE):
      @pl.when(jj > 0)
      def _(b=b):
        jprev = (jj - 1) * K_PIPE + b
        pltpu.make_async_copy(buf.at[b], aggr_sp.at[idx_v.at[jprev]],
                              semsc).wait()
        pltpu.make_async_copy(
            buf.at[b], erow_hbm.at[pl.ds(base + jprev * CK, CK)],
            semo).wait()

      def trow(ch, carry2, b=b):
        col = jnp.full((16,), ch, jnp.int32)
        for gseg in range(CK // 16):
          v = buft[p + b, ch, pl.ds(16 * gseg, 16)]
          plsc.store_scatter(buf.at[b], [lanes + (16 * gseg), col], v)
        return carry2

      lax.fori_loop(0, DE, trow, 0)
      pltpu.async_copy(buf.at[b], aggr_sp.at[idx_v.at[j0 + b]], semsc,
                       add=True)
      pltpu.async_copy(
          buf.at[b], erow_hbm.at[pl.ds(base + (j0 + b) * CK, CK)], semo)
    return carry

  lax.fori_loop(0, NG, group, 0)
  for b in range(K_PIPE):
    jprev = (NG - 1) * K_PIPE + b
    pltpu.make_async_copy(buf.at[b], aggr_sp.at[idx_v.at[jprev]],
                          semsc).wait()
    pltpu.make_async_copy(
        buf.at[b], erow_hbm.at[pl.ds(base + jprev * CK, CK)], semo).wait()
  plsc.subcore_barrier()
  pltpu.sync_copy(aggr_sp.at[pl.ds(s * NT, NT)],
                  part_hbm.at[c, pl.ds(s * NT, NT)])


_scatter_call = pl.kernel(
    _scatter_body,
    out_type=(jax.ShapeDtypeStruct((NC, N, DE), _f32),
              jax.ShapeDtypeStruct((E, DE), _f32)),
    mesh=plsc.VectorSubcoreMesh(core_axis_name="c", subcore_axis_name="s"),
    compiler_params=pltpu.CompilerParams(use_tc_tiling_on_sc=False,
                                         needs_layout_passes=False),
    scratch_types=[
        pltpu.VMEM((C, CK), jnp.int32),
        pltpu.VMEM((2 * K_PIPE, DE, CK), _f32),
        pltpu.VMEM((K_PIPE, CK, DE), _f32),
        pltpu.VMEM((NT, DE), _f32),
        pltpu.SemaphoreType.DMA,
        pltpu.SemaphoreType.DMA,
        pltpu.SemaphoreType.DMA,
        pltpu.VMEM_SHARED((N, DE), _f32),
    ],
)


BN = 5000
BE = 32000


def _proj_body(h_ref, wd_ref, ws_ref, pd_ref, ps_ref):
  h = h_ref[...]
  pd_ref[...] = _dot(h, wd_ref[...])
  ps_ref[...] = _dot(h, ws_ref[...])


def _proj(h, wd, ws):
  return pl.pallas_call(
      _proj_body,
      grid=(N // BN,),
      in_specs=[
          pl.BlockSpec((BN, D), lambda i: (i, 0)),
          pl.BlockSpec((D, HP), lambda i: (0, 0)),
          pl.BlockSpec((D, HP), lambda i: (0, 0)),
      ],
      out_specs=[
          pl.BlockSpec((BN, HP), lambda i: (i, 0)),
          pl.BlockSpec((BN, HP), lambda i: (i, 0)),
      ],
      out_shape=[
          jax.ShapeDtypeStruct((N, HP), _f32),
          jax.ShapeDtypeStruct((N, HP), _f32),
      ],
  )(h, wd, ws)


def _edge_body(gt_ref, eat_ref, wet_ref, b1_ref, w2t_ref, b2_ref, out_ref):
  z = gt_ref[...] + _dot(wet_ref[...], eat_ref[...]) + b1_ref[...]
  z = jnp.maximum(z, 0.0)
  out_ref[...] = _dot(w2t_ref[...], z) + b2_ref[...]


def _edge_mlp(gt, eat, wet, b1, w2t, b2):
  return pl.pallas_call(
      _edge_body,
      grid=(E // BE,),
      in_specs=[
          pl.BlockSpec((HP, BE), lambda i: (0, i)),
          pl.BlockSpec((DE, BE), lambda i: (0, i)),
          pl.BlockSpec((HP, DE), lambda i: (0, 0)),
          pl.BlockSpec((HP, 1), lambda i: (0, 0)),
          pl.BlockSpec((DE, HP), lambda i: (0, 0)),
          pl.BlockSpec((DE, 1), lambda i: (0, 0)),
      ],
      out_specs=pl.BlockSpec((DE, BE), lambda i: (0, i)),
      out_shape=jax.ShapeDtypeStruct((DE, E), _f32),
  )(gt, eat, wet, b1, w2t, b2)


def _node_body(h_ref, part_ref, w1h_ref, w1a_ref, b1_ref, w2_ref, b2_ref,
               wd_ref, ws_ref, hn_ref, pd_ref, ps_ref):
  h = h_ref[...]
  aggr = part_ref[0] + part_ref[1]
  u = jnp.maximum(_dot(h, w1h_ref[...]) + _dot(aggr, w1a_ref[...])
                  + b1_ref[...], 0.0)
  delta = _dot(u, w2_ref[...]) + b2_ref[...]
  hn = ALPHA * h + (1.0 - ALPHA) * jnp.maximum(delta, 0.0)
  hn_ref[...] = hn
  pd_ref[...] = _dot(hn, wd_ref[...])
  ps_ref[...] = _dot(hn, ws_ref[...])


def _node_mlp(h, part, w1h, w1a, b1, w2, b2, wd, ws):
  return pl.pallas_call(
      _node_body,
      grid=(N // BN,),
      in_specs=[
          pl.BlockSpec((BN, D), lambda i: (i, 0)),
          pl.BlockSpec((NC, BN, DE), lambda i: (0, i, 0)),
          pl.BlockSpec((D, H), lambda i: (0, 0)),
          pl.BlockSpec((DE, H), lambda i: (0, 0)),
          pl.BlockSpec((1, H), lambda i: (0, 0)),
          pl.BlockSpec((H, D), lambda i: (0, 0)),
          pl.BlockSpec((1, D), lambda i: (0, 0)),
          pl.BlockSpec((D, HP), lambda i: (0, 0)),
          pl.BlockSpec((D, HP), lambda i: (0, 0)),
      ],
      out_specs=[
          pl.BlockSpec((BN, D), lambda i: (i, 0)),
          pl.BlockSpec((BN, HP), lambda i: (i, 0)),
          pl.BlockSpec((BN, HP), lambda i: (i, 0)),
      ],
      out_shape=[
          jax.ShapeDtypeStruct((N, D), _f32),
          jax.ShapeDtypeStruct((N, HP), _f32),
          jax.ShapeDtypeStruct((N, HP), _f32),
      ],
  )(h, part, w1h, w1a, b1, w2, b2, wd, ws)


def kernel(h, edge_index, edge_attr, Wr1, br1, Wr2, br2, Wo1, bo1, Wo2, bo2):
  src = edge_index[0]
  dst = edge_index[1]
  dst3 = dst.reshape(NW, C, CK)
  src3 = src.reshape(NW, C, CK)

  pad_c = lambda m: jnp.pad(m, ((0, 0), (0, HP - H)))
  Wd = [pad_c(Wr1[l, :D]) for l in range(L)]
  Ws = [pad_c(Wr1[l, D:2 * D]) for l in range(L)]
  WeT = [pad_c(Wr1[l, 2 * D:]).T for l in range(L)]
  B1 = [jnp.pad(br1[l].reshape(H, 1), ((0, HP - H), (0, 0))) for l in range(L)]
  W2T = [jnp.pad(Wr2[l], ((0, HP - H), (0, 0))).T for l in range(L)]
  B2 = [br2[l].reshape(DE, 1) for l in range(L)]
  W1h = [Wo1[l, :D] for l in range(L)]
  W1a = [Wo1[l, D:] for l in range(L)]
  Bo1 = [bo1[l].reshape(1, H) for l in range(L)]
  Wo2l = [Wo2[l] for l in range(L)]
  Bo2 = [bo2[l].reshape(1, D) for l in range(L)]
  zeroW = jnp.zeros((D, HP), _f32)

  pd, ps = _proj(h, Wd[0], Ws[0])
  hs = [h]
  eas = [edge_attr]
  eat = edge_attr.T
  for l in range(L):
    gt = _gather_call(pd, ps, dst3, src3)
    et = _edge_mlp(gt, eat, WeT[l], B1[l], W2T[l], B2[l])
    part, e_row = _scatter_call(et, dst3)
    wd_next = Wd[l + 1] if l + 1 < L else zeroW
    ws_next = Ws[l + 1] if l + 1 < L else zeroW
    h, pd, ps = _node_mlp(h, part, W1h[l], W1a[l], Bo1[l], Wo2l[l], Bo2[l],
                          wd_next, ws_next)
    eat = et
    hs.append(h)
    eas.append(e_row)
  return (h, jnp.stack(hs), jnp.stack(eas))


# speedup vs baseline: 1.0193x; 1.0166x over previous
import functools

import jax
import jax.numpy as jnp
from jax import lax
from jax.experimental import pallas as pl
from jax.experimental.pallas import tpu as pltpu
from jax.experimental.pallas import tpu_sc as plsc

N = 10000
E = 320000
D = 128
DE = 16
H = 40
HP = 48
L = 3
ALPHA = 0.5

NC = 2
NS = 16
NW = NC * NS
ET = E // NW
CK = 80
C = ET // CK
NT = N // NS

_f32 = jnp.float32


def _dot(a, b):
  return jnp.dot(a, b, preferred_element_type=_f32)


K_PIPE = 5
CKG = 40
CG = ET // CKG
KG = 10


def _gather_body(pd_hbm, ps_hbm, dsti_hbm, srci_hbm, gt_hbm,
                 idxd_v, idxs_v, bufd, bufs, outt, semd, sems, semo):
  c = lax.axis_index("c")
  s = lax.axis_index("s")
  w = s * NC + c
  pltpu.sync_copy(dsti_hbm.at[w], idxd_v)
  pltpu.sync_copy(srci_hbm.at[w], idxs_v)
  base = w * ET
  lanes = lax.iota(jnp.int32, 16)

  NG = CG // KG

  def fire(jj):
    j0 = jj * KG
    p = (jj % 2) * KG
    for b in range(KG):
      pltpu.async_copy(pd_hbm.at[idxd_v.at[j0 + b]], bufd.at[p + b], semd)
      pltpu.async_copy(ps_hbm.at[idxs_v.at[j0 + b]], bufs.at[p + b], sems)

  fire(0)

  def group(jj, carry):
    @pl.when(jj + 1 < NG)
    def _():
      fire(jj + 1)

    j0 = jj * KG
    p = (jj % 2) * KG
    for b in range(KG):
      pltpu.make_async_copy(pd_hbm.at[idxd_v.at[j0 + b]], bufd.at[p + b],
                            semd).wait()
      pltpu.make_async_copy(ps_hbm.at[idxs_v.at[j0 + b]], bufs.at[p + b],
                            sems).wait()
    for b in range(KG):
      @pl.when(jj > 0)
      def _(b=b):
        jprev = (jj - 1) * KG + b
        pltpu.make_async_copy(
            outt.at[b], gt_hbm.at[:, pl.ds(base + jprev * CKG, CKG)],
            semo).wait()

      def addrow(r, carry2, b=b):
        col = jnp.full((16,), r, jnp.int32)
        for t in range(HP // 16):
          sl = pl.ds(t * 16, 16)
          v = bufd[p + b, r, sl] + bufs[p + b, r, sl]
          plsc.store_scatter(outt.at[b], [lanes + (16 * t), col], v)
        return carry2

      lax.fori_loop(0, CKG, addrow, 0)
      pltpu.async_copy(
          outt.at[b], gt_hbm.at[:, pl.ds(base + (j0 + b) * CKG, CKG)], semo)
    return carry

  lax.fori_loop(0, NG, group, 0)
  for b in range(KG):
    jprev = (NG - 1) * KG + b
    pltpu.make_async_copy(
        outt.at[b], gt_hbm.at[:, pl.ds(base + jprev * CKG, CKG)], semo).wait()


_gather_call = pl.kernel(
    _gather_body,
    out_type=jax.ShapeDtypeStruct((HP, E), _f32),
    mesh=plsc.VectorSubcoreMesh(core_axis_name="c", subcore_axis_name="s"),
    compiler_params=pltpu.CompilerParams(use_tc_tiling_on_sc=False,
                                         needs_layout_passes=False),
    scratch_types=[
        pltpu.VMEM((CG, CKG), jnp.int32),
        pltpu.VMEM((CG, CKG), jnp.int32),
        pltpu.VMEM((2 * KG, CKG, HP), _f32),
        pltpu.VMEM((2 * KG, CKG, HP), _f32),
        pltpu.VMEM((KG, HP, CKG), _f32),
        pltpu.SemaphoreType.DMA,
        pltpu.SemaphoreType.DMA,
        pltpu.SemaphoreType.DMA,
    ],
)


def _scatter_body(et_hbm, dsti_hbm, part_hbm, erow_hbm, idx_v, buft, buf,
                  zbuf, seml, semsc, semo, aggr_sp):
  c = lax.axis_index("c")
  s = lax.axis_index("s")
  w = s * NC + c

  def zrow(i, carry):
    zbuf[i] = jnp.zeros((16,), _f32)
    return carry

  lax.fori_loop(0, NT, zrow, 0)
  pltpu.sync_copy(zbuf, aggr_sp.at[pl.ds(s * NT, NT)])
  pltpu.sync_copy(dsti_hbm.at[w], idx_v)
  plsc.subcore_barrier()

  base = w * ET
  lanes = lax.iota(jnp.int32, 16)
  NG = C // K_PIPE

  def fire(jj):
    j0 = jj * K_PIPE
    p = (jj % 2) * K_PIPE
    for b in range(K_PIPE):
      pltpu.async_copy(et_hbm.at[:, pl.ds(base + (j0 + b) * CK, CK)],
                       buft.at[p + b], seml)

  fire(0)

  def group(jj, carry):
    @pl.when(jj + 1 < NG)
    def _():
      fire(jj + 1)

    j0 = jj * K_PIPE
    p = (jj % 2) * K_PIPE
    for b in range(K_PIPE):
      pltpu.make_async_copy(et_hbm.at[:, pl.ds(base + (j0 + b) * CK, CK)],
                            buft.at[p + b], seml).wait()
    for b in range(K_PIPE):
      @pl.when(jj > 0)
      def _(b=b):
        jprev = (jj - 1) * K_PIPE + b
        pltpu.make_async_copy(buf.at[b], aggr_sp.at[idx_v.at[jprev]],
                              semsc).wait()
        pltpu.make_async_copy(
            buf.at[b], erow_hbm.at[pl.ds(base + jprev * CK, CK)],
            semo).wait()

      def trow(ch, carry2, b=b):
        col = jnp.full((16,), ch, jnp.int32)
        for gseg in range(CK // 16):
          v = buft[p + b, ch, pl.ds(16 * gseg, 16)]
          plsc.store_scatter(buf.at[b], [lanes + (16 * gseg), col], v)
        return carry2

      lax.fori_loop(0, DE, trow, 0)
      pltpu.async_copy(buf.at[b], aggr_sp.at[idx_v.at[j0 + b]], semsc,
                       add=True)
      pltpu.async_copy(
          buf.at[b], erow_hbm.at[pl.ds(base + (j0 + b) * CK, CK)], semo)
    return carry

  lax.fori_loop(0, NG, group, 0)
  for b in range(K_PIPE):
    jprev = (NG - 1) * K_PIPE + b
    pltpu.make_async_copy(buf.at[b], aggr_sp.at[idx_v.at[jprev]],
                          semsc).wait()
    pltpu.make_async_copy(
        buf.at[b], erow_hbm.at[pl.ds(base + jprev * CK, CK)], semo).wait()
  plsc.subcore_barrier()
  pltpu.sync_copy(aggr_sp.at[pl.ds(s * NT, NT)],
                  part_hbm.at[c, pl.ds(s * NT, NT)])


_scatter_call = pl.kernel(
    _scatter_body,
    out_type=(jax.ShapeDtypeStruct((NC, N, DE), _f32),
              jax.ShapeDtypeStruct((E, DE), _f32)),
    mesh=plsc.VectorSubcoreMesh(core_axis_name="c", subcore_axis_name="s"),
    compiler_params=pltpu.CompilerParams(use_tc_tiling_on_sc=False,
                                         needs_layout_passes=False),
    scratch_types=[
        pltpu.VMEM((C, CK), jnp.int32),
        pltpu.VMEM((2 * K_PIPE, DE, CK), _f32),
        pltpu.VMEM((K_PIPE, CK, DE), _f32),
        pltpu.VMEM((NT, DE), _f32),
        pltpu.SemaphoreType.DMA,
        pltpu.SemaphoreType.DMA,
        pltpu.SemaphoreType.DMA,
        pltpu.VMEM_SHARED((N, DE), _f32),
    ],
)


BN = 5000
BE = 32000


def _proj_body(h_ref, wd_ref, ws_ref, pd_ref, ps_ref):
  h = h_ref[...]
  pd_ref[...] = _dot(h, wd_ref[...])
  ps_ref[...] = _dot(h, ws_ref[...])


def _proj(h, wd, ws):
  return pl.pallas_call(
      _proj_body,
      grid=(N // BN,),
      in_specs=[
          pl.BlockSpec((BN, D), lambda i: (i, 0)),
          pl.BlockSpec((D, HP), lambda i: (0, 0)),
          pl.BlockSpec((D, HP), lambda i: (0, 0)),
      ],
      out_specs=[
          pl.BlockSpec((BN, HP), lambda i: (i, 0)),
          pl.BlockSpec((BN, HP), lambda i: (i, 0)),
      ],
      out_shape=[
          jax.ShapeDtypeStruct((N, HP), _f32),
          jax.ShapeDtypeStruct((N, HP), _f32),
      ],
  )(h, wd, ws)


def _edge_body(gt_ref, eat_ref, wet_ref, b1_ref, w2t_ref, b2_ref, out_ref):
  z = gt_ref[...] + _dot(wet_ref[...], eat_ref[...]) + b1_ref[...]
  z = jnp.maximum(z, 0.0)
  out_ref[...] = _dot(w2t_ref[...], z) + b2_ref[...]


def _edge_mlp(gt, eat, wet, b1, w2t, b2):
  return pl.pallas_call(
      _edge_body,
      grid=(E // BE,),
      in_specs=[
          pl.BlockSpec((HP, BE), lambda i: (0, i)),
          pl.BlockSpec((DE, BE), lambda i: (0, i)),
          pl.BlockSpec((HP, DE), lambda i: (0, 0)),
          pl.BlockSpec((HP, 1), lambda i: (0, 0)),
          pl.BlockSpec((DE, HP), lambda i: (0, 0)),
          pl.BlockSpec((DE, 1), lambda i: (0, 0)),
      ],
      out_specs=pl.BlockSpec((DE, BE), lambda i: (0, i)),
      out_shape=jax.ShapeDtypeStruct((DE, E), _f32),
  )(gt, eat, wet, b1, w2t, b2)


def _node_body(h_ref, part_ref, w1h_ref, w1a_ref, b1_ref, w2_ref, b2_ref,
               wd_ref, ws_ref, hn_ref, pd_ref, ps_ref):
  h = h_ref[...]
  aggr = part_ref[0] + part_ref[1]
  u = jnp.maximum(_dot(h, w1h_ref[...]) + _dot(aggr, w1a_ref[...])
                  + b1_ref[...], 0.0)
  delta = _dot(u, w2_ref[...]) + b2_ref[...]
  hn = ALPHA * h + (1.0 - ALPHA) * jnp.maximum(delta, 0.0)
  hn_ref[...] = hn
  pd_ref[...] = _dot(hn, wd_ref[...])
  ps_ref[...] = _dot(hn, ws_ref[...])


def _node_mlp(h, part, w1h, w1a, b1, w2, b2, wd, ws):
  return pl.pallas_call(
      _node_body,
      grid=(N // BN,),
      in_specs=[
          pl.BlockSpec((BN, D), lambda i: (i, 0)),
          pl.BlockSpec((NC, BN, DE), lambda i: (0, i, 0)),
          pl.BlockSpec((D, H), lambda i: (0, 0)),
          pl.BlockSpec((DE, H), lambda i: (0, 0)),
          pl.BlockSpec((1, H), lambda i: (0, 0)),
          pl.BlockSpec((H, D), lambda i: (0, 0)),
          pl.BlockSpec((1, D), lambda i: (0, 0)),
          pl.BlockSpec((D, HP), lambda i: (0, 0)),
          pl.BlockSpec((D, HP), lambda i: (0, 0)),
      ],
      out_specs=[
          pl.BlockSpec((BN, D), lambda i: (i, 0)),
          pl.BlockSpec((BN, HP), lambda i: (i, 0)),
          pl.BlockSpec((BN, HP), lambda i: (i, 0)),
      ],
      out_shape=[
          jax.ShapeDtypeStruct((N, D), _f32),
          jax.ShapeDtypeStruct((N, HP), _f32),
          jax.ShapeDtypeStruct((N, HP), _f32),
      ],
  )(h, part, w1h, w1a, b1, w2, b2, wd, ws)


def kernel(h, edge_index, edge_attr, Wr1, br1, Wr2, br2, Wo1, bo1, Wo2, bo2):
  src = edge_index[0]
  dst = edge_index[1]
  dst3 = dst.reshape(NW, C, CK)
  src3 = src.reshape(NW, C, CK)
  dst3g = dst.reshape(NW, CG, CKG)
  src3g = src.reshape(NW, CG, CKG)

  pad_c = lambda m: jnp.pad(m, ((0, 0), (0, HP - H)))
  Wd = [pad_c(Wr1[l, :D]) for l in range(L)]
  Ws = [pad_c(Wr1[l, D:2 * D]) for l in range(L)]
  WeT = [pad_c(Wr1[l, 2 * D:]).T for l in range(L)]
  B1 = [jnp.pad(br1[l].reshape(H, 1), ((0, HP - H), (0, 0))) for l in range(L)]
  W2T = [jnp.pad(Wr2[l], ((0, HP - H), (0, 0))).T for l in range(L)]
  B2 = [br2[l].reshape(DE, 1) for l in range(L)]
  W1h = [Wo1[l, :D] for l in range(L)]
  W1a = [Wo1[l, D:] for l in range(L)]
  Bo1 = [bo1[l].reshape(1, H) for l in range(L)]
  Wo2l = [Wo2[l] for l in range(L)]
  Bo2 = [bo2[l].reshape(1, D) for l in range(L)]
  zeroW = jnp.zeros((D, HP), _f32)

  pd, ps = _proj(h, Wd[0], Ws[0])
  hs = [h]
  eas = [edge_attr]
  eat = edge_attr.T
  for l in range(L):
    gt = _gather_call(pd, ps, dst3g, src3g)
    et = _edge_mlp(gt, eat, WeT[l], B1[l], W2T[l], B2[l])
    part, e_row = _scatter_call(et, dst3)
    wd_next = Wd[l + 1] if l + 1 < L else zeroW
    ws_next = Ws[l + 1] if l + 1 < L else zeroW
    h, pd, ps = _node_mlp(h, part, W1h[l], W1a[l], Bo1[l], Wo2l[l], Bo2[l],
                          wd_next, ws_next)
    eat = et
    hs.append(h)
    eas.append(e_row)
  return (h, jnp.stack(hs), jnp.stack(eas))


# speedup vs baseline: 1.0290x; 1.0095x over previous
import functools

import jax
import jax.numpy as jnp
from jax import lax
from jax.experimental import pallas as pl
from jax.experimental.pallas import tpu as pltpu
from jax.experimental.pallas import tpu_sc as plsc

N = 10000
E = 320000
D = 128
DE = 16
H = 40
HP = 48
L = 3
ALPHA = 0.5

NC = 2
NS = 16
NW = NC * NS
ET = E // NW
CK = 40
C = ET // CK
NT = N // NS

_f32 = jnp.float32


def _dot(a, b):
  return jnp.dot(a, b, preferred_element_type=_f32)


K_PIPE = 10
CKG = 40
CG = ET // CKG
KG = 10


def _gather_body(pd_hbm, ps_hbm, dsti_hbm, srci_hbm, gt_hbm,
                 idxd_v, idxs_v, bufd, bufs, outt, semd, sems, semo):
  c = lax.axis_index("c")
  s = lax.axis_index("s")
  w = s * NC + c
  pltpu.sync_copy(dsti_hbm.at[w], idxd_v)
  pltpu.sync_copy(srci_hbm.at[w], idxs_v)
  base = w * ET
  lanes = lax.iota(jnp.int32, 16)

  NG = CG // KG

  def fire(jj):
    j0 = jj * KG
    p = (jj % 2) * KG
    for b in range(KG):
      pltpu.async_copy(pd_hbm.at[idxd_v.at[j0 + b]], bufd.at[p + b], semd)
      pltpu.async_copy(ps_hbm.at[idxs_v.at[j0 + b]], bufs.at[p + b], sems)

  fire(0)

  def group(jj, carry):
    @pl.when(jj + 1 < NG)
    def _():
      fire(jj + 1)

    j0 = jj * KG
    p = (jj % 2) * KG
    for b in range(KG):
      pltpu.make_async_copy(pd_hbm.at[idxd_v.at[j0 + b]], bufd.at[p + b],
                            semd).wait()
      pltpu.make_async_copy(ps_hbm.at[idxs_v.at[j0 + b]], bufs.at[p + b],
                            sems).wait()
    for b in range(KG):
      @pl.when(jj > 0)
      def _(b=b):
        jprev = (jj - 1) * KG + b
        pltpu.make_async_copy(
            outt.at[b], gt_hbm.at[:, pl.ds(base + jprev * CKG, CKG)],
            semo).wait()

      def addrow(r, carry2, b=b):
        col = jnp.full((16,), r, jnp.int32)
        for t in range(HP // 16):
          sl = pl.ds(t * 16, 16)
          v = bufd[p + b, r, sl] + bufs[p + b, r, sl]
          plsc.store_scatter(outt.at[b], [lanes + (16 * t), col], v)
        return carry2

      lax.fori_loop(0, CKG, addrow, 0)
      pltpu.async_copy(
          outt.at[b], gt_hbm.at[:, pl.ds(base + (j0 + b) * CKG, CKG)], semo)
    return carry

  lax.fori_loop(0, NG, group, 0)
  for b in range(KG):
    jprev = (NG - 1) * KG + b
    pltpu.make_async_copy(
        outt.at[b], gt_hbm.at[:, pl.ds(base + jprev * CKG, CKG)], semo).wait()


_gather_call = pl.kernel(
    _gather_body,
    out_type=jax.ShapeDtypeStruct((HP, E), _f32),
    mesh=plsc.VectorSubcoreMesh(core_axis_name="c", subcore_axis_name="s"),
    compiler_params=pltpu.CompilerParams(use_tc_tiling_on_sc=False,
                                         needs_layout_passes=False),
    scratch_types=[
        pltpu.VMEM((CG, CKG), jnp.int32),
        pltpu.VMEM((CG, CKG), jnp.int32),
        pltpu.VMEM((2 * KG, CKG, HP), _f32),
        pltpu.VMEM((2 * KG, CKG, HP), _f32),
        pltpu.VMEM((KG, HP, CKG), _f32),
        pltpu.SemaphoreType.DMA,
        pltpu.SemaphoreType.DMA,
        pltpu.SemaphoreType.DMA,
    ],
)


def _scatter_body(et_hbm, dsti_hbm, part_hbm, erow_hbm, idx_v, buft, buf,
                  zbuf, seml, semsc, semo, aggr_sp):
  c = lax.axis_index("c")
  s = lax.axis_index("s")
  w = s * NC + c

  def zrow(i, carry):
    zbuf[i] = jnp.zeros((16,), _f32)
    return carry

  lax.fori_loop(0, NT, zrow, 0)
  pltpu.sync_copy(zbuf, aggr_sp.at[pl.ds(s * NT, NT)])
  pltpu.sync_copy(dsti_hbm.at[w], idx_v)
  plsc.subcore_barrier()

  base = w * ET
  lanes = lax.iota(jnp.int32, 16)
  NG = C // K_PIPE

  def fire(jj):
    j0 = jj * K_PIPE
    p = (jj % 2) * K_PIPE
    for b in range(K_PIPE):
      pltpu.async_copy(et_hbm.at[:, pl.ds(base + (j0 + b) * CK, CK)],
                       buft.at[p + b], seml)

  fire(0)

  def group(jj, carry):
    @pl.when(jj + 1 < NG)
    def _():
      fire(jj + 1)

    j0 = jj * K_PIPE
    p = (jj % 2) * K_PIPE
    for b in range(K_PIPE):
      pltpu.make_async_copy(et_hbm.at[:, pl.ds(base + (j0 + b) * CK, CK)],
                            buft.at[p + b], seml).wait()
    for b in range(K_PIPE):
      @pl.when(jj > 0)
      def _(b=b):
        jprev = (jj - 1) * K_PIPE + b
        pltpu.make_async_copy(buf.at[b], aggr_sp.at[idx_v.at[jprev]],
                              semsc).wait()
        pltpu.make_async_copy(
            buf.at[b], erow_hbm.at[pl.ds(base + jprev * CK, CK)],
            semo).wait()

      def trow(ch, carry2, b=b):
        col = jnp.full((16,), ch, jnp.int32)
        for gseg in range(CK // 16):
          v = buft[p + b, ch, pl.ds(16 * gseg, 16)]
          plsc.store_scatter(buf.at[b], [lanes + (16 * gseg), col], v)
        return carry2

      lax.fori_loop(0, DE, trow, 0)
      pltpu.async_copy(buf.at[b], aggr_sp.at[idx_v.at[j0 + b]], semsc,
                       add=True)
      pltpu.async_copy(
          buf.at[b], erow_hbm.at[pl.ds(base + (j0 + b) * CK, CK)], semo)
    return carry

  lax.fori_loop(0, NG, group, 0)
  for b in range(K_PIPE):
    jprev = (NG - 1) * K_PIPE + b
    pltpu.make_async_copy(buf.at[b], aggr_sp.at[idx_v.at[jprev]],
                          semsc).wait()
    pltpu.make_async_copy(
        buf.at[b], erow_hbm.at[pl.ds(base + jprev * CK, CK)], semo).wait()
  plsc.subcore_barrier()
  pltpu.sync_copy(aggr_sp.at[pl.ds(s * NT, NT)],
                  part_hbm.at[c, pl.ds(s * NT, NT)])


_scatter_call = pl.kernel(
    _scatter_body,
    out_type=(jax.ShapeDtypeStruct((NC, N, DE), _f32),
              jax.ShapeDtypeStruct((E, DE), _f32)),
    mesh=plsc.VectorSubcoreMesh(core_axis_name="c", subcore_axis_name="s"),
    compiler_params=pltpu.CompilerParams(use_tc_tiling_on_sc=False,
                                         needs_layout_passes=False),
    scratch_types=[
        pltpu.VMEM((C, CK), jnp.int32),
        pltpu.VMEM((2 * K_PIPE, DE, CK), _f32),
        pltpu.VMEM((K_PIPE, CK, DE), _f32),
        pltpu.VMEM((NT, DE), _f32),
        pltpu.SemaphoreType.DMA,
        pltpu.SemaphoreType.DMA,
        pltpu.SemaphoreType.DMA,
        pltpu.VMEM_SHARED((N, DE), _f32),
    ],
)


BN = 5000
BE = 32000


def _proj_body(h_ref, wd_ref, ws_ref, pd_ref, ps_ref):
  h = h_ref[...]
  pd_ref[...] = _dot(h, wd_ref[...])
  ps_ref[...] = _dot(h, ws_ref[...])


def _proj(h, wd, ws):
  return pl.pallas_call(
      _proj_body,
      grid=(N // BN,),
      in_specs=[
          pl.BlockSpec((BN, D), lambda i: (i, 0)),
          pl.BlockSpec((D, HP), lambda i: (0, 0)),
          pl.BlockSpec((D, HP), lambda i: (0, 0)),
      ],
      out_specs=[
          pl.BlockSpec((BN, HP), lambda i: (i, 0)),
          pl.BlockSpec((BN, HP), lambda i: (i, 0)),
      ],
      out_shape=[
          jax.ShapeDtypeStruct((N, HP), _f32),
          jax.ShapeDtypeStruct((N, HP), _f32),
      ],
  )(h, wd, ws)


def _edge_body(gt_ref, eat_ref, wet_ref, b1_ref, w2t_ref, b2_ref, out_ref):
  z = gt_ref[...] + _dot(wet_ref[...], eat_ref[...]) + b1_ref[...]
  z = jnp.maximum(z, 0.0)
  out_ref[...] = _dot(w2t_ref[...], z) + b2_ref[...]


def _edge_mlp(gt, eat, wet, b1, w2t, b2):
  return pl.pallas_call(
      _edge_body,
      grid=(E // BE,),
      in_specs=[
          pl.BlockSpec((HP, BE), lambda i: (0, i)),
          pl.BlockSpec((DE, BE), lambda i: (0, i)),
          pl.BlockSpec((HP, DE), lambda i: (0, 0)),
          pl.BlockSpec((HP, 1), lambda i: (0, 0)),
          pl.BlockSpec((DE, HP), lambda i: (0, 0)),
          pl.BlockSpec((DE, 1), lambda i: (0, 0)),
      ],
      out_specs=pl.BlockSpec((DE, BE), lambda i: (0, i)),
      out_shape=jax.ShapeDtypeStruct((DE, E), _f32),
  )(gt, eat, wet, b1, w2t, b2)


def _node_body(h_ref, part_ref, w1h_ref, w1a_ref, b1_ref, w2_ref, b2_ref,
               wd_ref, ws_ref, hn_ref, pd_ref, ps_ref):
  h = h_ref[...]
  aggr = part_ref[0] + part_ref[1]
  u = jnp.maximum(_dot(h, w1h_ref[...]) + _dot(aggr, w1a_ref[...])
                  + b1_ref[...], 0.0)
  delta = _dot(u, w2_ref[...]) + b2_ref[...]
  hn = ALPHA * h + (1.0 - ALPHA) * jnp.maximum(delta, 0.0)
  hn_ref[...] = hn
  pd_ref[...] = _dot(hn, wd_ref[...])
  ps_ref[...] = _dot(hn, ws_ref[...])


def _node_mlp(h, part, w1h, w1a, b1, w2, b2, wd, ws):
  return pl.pallas_call(
      _node_body,
      grid=(N // BN,),
      in_specs=[
          pl.BlockSpec((BN, D), lambda i: (i, 0)),
          pl.BlockSpec((NC, BN, DE), lambda i: (0, i, 0)),
          pl.BlockSpec((D, H), lambda i: (0, 0)),
          pl.BlockSpec((DE, H), lambda i: (0, 0)),
          pl.BlockSpec((1, H), lambda i: (0, 0)),
          pl.BlockSpec((H, D), lambda i: (0, 0)),
          pl.BlockSpec((1, D), lambda i: (0, 0)),
          pl.BlockSpec((D, HP), lambda i: (0, 0)),
          pl.BlockSpec((D, HP), lambda i: (0, 0)),
      ],
      out_specs=[
          pl.BlockSpec((BN, D), lambda i: (i, 0)),
          pl.BlockSpec((BN, HP), lambda i: (i, 0)),
          pl.BlockSpec((BN, HP), lambda i: (i, 0)),
      ],
      out_shape=[
          jax.ShapeDtypeStruct((N, D), _f32),
          jax.ShapeDtypeStruct((N, HP), _f32),
          jax.ShapeDtypeStruct((N, HP), _f32),
      ],
  )(h, part, w1h, w1a, b1, w2, b2, wd, ws)


def kernel(h, edge_index, edge_attr, Wr1, br1, Wr2, br2, Wo1, bo1, Wo2, bo2):
  src = edge_index[0]
  dst = edge_index[1]
  dst3 = dst.reshape(NW, C, CK)
  src3 = src.reshape(NW, C, CK)
  dst3g = dst.reshape(NW, CG, CKG)
  src3g = src.reshape(NW, CG, CKG)

  pad_c = lambda m: jnp.pad(m, ((0, 0), (0, HP - H)))
  Wd = [pad_c(Wr1[l, :D]) for l in range(L)]
  Ws = [pad_c(Wr1[l, D:2 * D]) for l in range(L)]
  WeT = [pad_c(Wr1[l, 2 * D:]).T for l in range(L)]
  B1 = [jnp.pad(br1[l].reshape(H, 1), ((0, HP - H), (0, 0))) for l in range(L)]
  W2T = [jnp.pad(Wr2[l], ((0, HP - H), (0, 0))).T for l in range(L)]
  B2 = [br2[l].reshape(DE, 1) for l in range(L)]
  W1h = [Wo1[l, :D] for l in range(L)]
  W1a = [Wo1[l, D:] for l in range(L)]
  Bo1 = [bo1[l].reshape(1, H) for l in range(L)]
  Wo2l = [Wo2[l] for l in range(L)]
  Bo2 = [bo2[l].reshape(1, D) for l in range(L)]
  zeroW = jnp.zeros((D, HP), _f32)

  pd, ps = _proj(h, Wd[0], Ws[0])
  hs = [h]
  eas = [edge_attr]
  eat = edge_attr.T
  for l in range(L):
    gt = _gather_call(pd, ps, dst3g, src3g)
    et = _edge_mlp(gt, eat, WeT[l], B1[l], W2T[l], B2[l])
    part, e_row = _scatter_call(et, dst3)
    wd_next = Wd[l + 1] if l + 1 < L else zeroW
    ws_next = Ws[l + 1] if l + 1 < L else zeroW
    h, pd, ps = _node_mlp(h, part, W1h[l], W1a[l], Bo1[l], Wo2l[l], Bo2[l],
                          wd_next, ws_next)
    eat = et
    hs.append(h)
    eas.append(e_row)
  return (h, jnp.stack(hs), jnp.stack(eas))
